# Initial kernel scaffold; baseline (speedup 1.0000x reference)
import jax
import jax.numpy as jnp
from jax.experimental import pallas as pl


def kernel(theta, bow, word_inputs, W1, b1, gamma, beta, W2, b2, embedding):
    raise NotImplementedError("write your pallas kernel here")


# speedup vs baseline: 2.1152x; 2.1152x over previous
import functools
import math

import jax
import jax.numpy as jnp
from jax import lax
from jax.experimental import pallas as pl
from jax.experimental.pallas import tpu as pltpu
from jax.experimental.pallas import tpu_sc as plsc

_N_TOPIC = 256
_V_DIM = 10000
_HID = 2048
_EMB = 300
_K = _N_TOPIC + _V_DIM

_BH = 256
_GRID = _HID // _BH


def _mlp_body(p_ref, w1_ref, scale_ref, shift_ref, w2_ref, b2_ref, out_ref):
    i = pl.program_id(0)
    h = lax.dot_general(
        p_ref[...], w1_ref[...], (((1,), (1,)), ((), ())),
        preferred_element_type=jnp.float32,
    )
    h = h * scale_ref[...] + shift_ref[...]
    h = jnp.where(h > 0.0, h, 0.01 * h)
    part = jnp.sum(h * w2_ref[...], axis=1, keepdims=True)

    @pl.when(i == 0)
    def _init():
        out_ref[...] = b2_ref[...] + part

    @pl.when(i > 0)
    def _acc():
        out_ref[...] += part


_mlp_call = pl.pallas_call(
    _mlp_body,
    grid=(_GRID,),
    in_specs=[
        pl.BlockSpec((1, _K), lambda i: (0, 0)),
        pl.BlockSpec((_BH, _K), lambda i: (i, 0)),
        pl.BlockSpec((1, _BH), lambda i: (0, i)),
        pl.BlockSpec((1, _BH), lambda i: (0, i)),
        pl.BlockSpec((1, _BH), lambda i: (0, i)),
        pl.BlockSpec((1, 1), lambda i: (0, 0)),
    ],
    out_specs=pl.BlockSpec((1, 1), lambda i: (0, 0)),
    out_shape=jax.ShapeDtypeStruct((1, 1), jnp.float32),
)

_NW = 32
_BPW = 320
_PAD_N = _NW * _BPW
_GB = 80
_GCH = _BPW // _GB
_NCH = 19
_OUTW = 16 * _NCH


def _embbag_body(idx_hbm, w_hbm, table_hbm, out_hbm, idx_v, w_v, c0_v, c1_v, c2_v,
                 acc_v, sem, sem2):
    wid = lax.axis_index("s") * 2 + lax.axis_index("c")
    base = wid * _BPW
    pltpu.sync_copy(idx_hbm.at[pl.ds(base, _BPW)], idx_v)
    pltpu.sync_copy(w_hbm.at[pl.ds(wid * (_BPW // 8), _BPW // 8)], w_v)
    copies = []
    for g in range(_GCH):
        isl = idx_v.at[pl.ds(g * _GB, _GB)]
        dsl = pl.ds(g * _GB, _GB)
        copies.append(pltpu.async_copy(
            table_hbm.at[isl, pl.ds(0, 128)], c0_v.at[dsl], sem))
        copies.append(pltpu.async_copy(
            table_hbm.at[isl, pl.ds(128, 128)], c1_v.at[dsl], sem))

    def tail_group(j, carry):
        idxvec = idx_v[pl.ds(j * 16, 16)]
        for k in range(16):
            pltpu.async_copy(
                table_hbm.at[idxvec[k], pl.ds(256, 44)],
                c2_v.at[j * 16 + k, pl.ds(0, 44)], sem2)
        for k in range(16):
            pltpu.make_async_copy(
                table_hbm.at[0, pl.ds(256, 44)],
                c2_v.at[0, pl.ds(0, 44)], sem2).wait()
        return carry

    lax.fori_loop(0, _BPW // 16, tail_group, 0)

    for cp in copies:
        cp.wait()

    tail_mask = lax.iota(jnp.int32, 16) < 12

    def row_step(i, accs):
        w_splat = w_v[i // 8, pl.ds((i % 8) * 16, 16)]
        nxt = []
        for c in range(8):
            nxt.append(accs[c] + w_splat * c0_v[i, pl.ds(c * 16, 16)])
        for c in range(8):
            nxt.append(accs[8 + c] + w_splat * c1_v[i, pl.ds(c * 16, 16)])
        nxt.append(accs[16] + w_splat * c2_v[i, pl.ds(0, 16)])
        nxt.append(accs[17] + w_splat * c2_v[i, pl.ds(16, 16)])
        tail = c2_v[i, pl.ds(32, 16)]
        nxt.append(accs[_NCH - 1] + jnp.where(tail_mask, w_splat * tail, 0.0))
        return tuple(nxt)

    accs = lax.fori_loop(
        0, _BPW, row_step,
        tuple(jnp.zeros((16,), jnp.float32) for _ in range(_NCH)),
    )
    for c in range(_NCH):
        acc_v[pl.ds(c * 16, 16)] = accs[c]
    pltpu.sync_copy(acc_v, out_hbm.at[wid])


@functools.lru_cache(maxsize=1)
def _build_embbag():
    return functools.partial(
        pl.kernel,
        out_type=jax.ShapeDtypeStruct((_NW, _OUTW), jnp.float32),
        mesh=plsc.VectorSubcoreMesh(core_axis_name="c", subcore_axis_name="s"),
        scratch_types=[
            pltpu.VMEM((_BPW,), jnp.int32),
            pltpu.VMEM((_BPW // 8, 128), jnp.float32),
            pltpu.VMEM((_BPW, 128), jnp.float32),
            pltpu.VMEM((_BPW, 128), jnp.float32),
            pltpu.VMEM((_BPW, 48), jnp.float32),
            pltpu.VMEM((_OUTW,), jnp.float32),
            pltpu.SemaphoreType.DMA,
            pltpu.SemaphoreType.DMA,
        ],
    )(_embbag_body)


def kernel(theta, bow, word_inputs, W1, b1, gamma, beta, W2, b2, embedding):
    inv = 1.0 / math.sqrt(1.0 + 1e-5)
    p = jnp.concatenate([theta, bow]).reshape(1, _K)
    scale = (gamma * inv).reshape(1, _HID)
    shift = (b1 * gamma * inv + beta).reshape(1, _HID)
    score = _mlp_call(p, W1, scale, shift, W2.reshape(1, _HID), b2.reshape(1, 1))

    pad = _PAD_N - _V_DIM
    idx_p = jnp.concatenate([word_inputs, jnp.zeros((pad,), jnp.int32)])
    w_p = jnp.concatenate([bow, jnp.zeros((pad,), jnp.float32)])
    w_p = jnp.broadcast_to(
        w_p.reshape(_PAD_N // 8, 8, 1), (_PAD_N // 8, 8, 16)
    ).reshape(_PAD_N // 8, 128)
    parts = _build_embbag()(idx_p, w_p, embedding)
    s = jnp.sum(parts, axis=0)
    return score.reshape(1), s[:_EMB]


# speedup vs baseline: 2.8330x; 1.3394x over previous
import functools
import math

import jax
import jax.numpy as jnp
from jax import lax
from jax.experimental import pallas as pl
from jax.experimental.pallas import tpu as pltpu
from jax.experimental.pallas import tpu_sc as plsc

_N_TOPIC = 256
_V_DIM = 10000
_HID = 2048
_EMB = 300
_K = _N_TOPIC + _V_DIM

_BH = 256
_GRID = _HID // _BH


def _mlp_body(p_ref, w1t_ref, scale_ref, shift_ref, w2_ref, b2_ref, out_ref):
    i = pl.program_id(0)
    h = lax.dot_general(
        p_ref[...], w1t_ref[...], (((1,), (0,)), ((), ())),
        preferred_element_type=jnp.float32,
    )
    h = h * scale_ref[...] + shift_ref[...]
    h = jnp.where(h > 0.0, h, 0.01 * h)
    part = jnp.sum(h * w2_ref[...], axis=1, keepdims=True)

    @pl.when(i == 0)
    def _init():
        out_ref[...] = b2_ref[...] + part

    @pl.when(i > 0)
    def _acc():
        out_ref[...] += part


_mlp_call = pl.pallas_call(
    _mlp_body,
    grid=(_GRID,),
    in_specs=[
        pl.BlockSpec((1, _K), lambda i: (0, 0)),
        pl.BlockSpec((_K, _BH), lambda i: (0, i)),
        pl.BlockSpec((1, _BH), lambda i: (0, i)),
        pl.BlockSpec((1, _BH), lambda i: (0, i)),
        pl.BlockSpec((1, _BH), lambda i: (0, i)),
        pl.BlockSpec((1, 1), lambda i: (0, 0)),
    ],
    out_specs=pl.BlockSpec((1, 1), lambda i: (0, 0)),
    out_shape=jax.ShapeDtypeStruct((1, 1), jnp.float32),
)

_NW = 32
_BPW = 320
_PAD_N = _NW * _BPW
_GB = 80
_GCH = _BPW // _GB
_NCH = 19
_OUTW = 16 * _NCH


def _embbag_body(idx_hbm, w_hbm, table_hbm, out_hbm, idx_v, w_v, c0_v, c1_v, c2_v,
                 acc_v, sem, sem2):
    wid = lax.axis_index("s") * 2 + lax.axis_index("c")
    base = wid * _BPW
    pltpu.sync_copy(idx_hbm.at[pl.ds(base, _BPW)], idx_v)
    pltpu.sync_copy(w_hbm.at[pl.ds(wid * (_BPW // 8), _BPW // 8)], w_v)
    copies = []
    for g in range(_GCH):
        isl = idx_v.at[pl.ds(g * _GB, _GB)]
        dsl = pl.ds(g * _GB, _GB)
        copies.append(pltpu.async_copy(
            table_hbm.at[isl, pl.ds(0, 128)], c0_v.at[dsl], sem))
        copies.append(pltpu.async_copy(
            table_hbm.at[isl, pl.ds(128, 128)], c1_v.at[dsl], sem))

    def tail_group(j, carry):
        idxvec = idx_v[pl.ds(j * 16, 16)]
        for k in range(16):
            pltpu.async_copy(
                table_hbm.at[idxvec[k], pl.ds(256, 44)],
                c2_v.at[j * 16 + k, pl.ds(0, 44)], sem2)
        for k in range(16):
            pltpu.make_async_copy(
                table_hbm.at[0, pl.ds(256, 44)],
                c2_v.at[0, pl.ds(0, 44)], sem2).wait()
        return carry

    lax.fori_loop(0, _BPW // 16, tail_group, 0)

    for cp in copies:
        cp.wait()

    tail_mask = lax.iota(jnp.int32, 16) < 12

    def row_step(i, accs):
        w_splat = w_v[i // 8, pl.ds((i % 8) * 16, 16)]
        nxt = []
        for c in range(8):
            nxt.append(accs[c] + w_splat * c0_v[i, pl.ds(c * 16, 16)])
        for c in range(8):
            nxt.append(accs[8 + c] + w_splat * c1_v[i, pl.ds(c * 16, 16)])
        nxt.append(accs[16] + w_splat * c2_v[i, pl.ds(0, 16)])
        nxt.append(accs[17] + w_splat * c2_v[i, pl.ds(16, 16)])
        tail = c2_v[i, pl.ds(32, 16)]
        nxt.append(accs[_NCH - 1] + jnp.where(tail_mask, w_splat * tail, 0.0))
        return tuple(nxt)

    accs = lax.fori_loop(
        0, _BPW, row_step,
        tuple(jnp.zeros((16,), jnp.float32) for _ in range(_NCH)),
    )
    for c in range(_NCH):
        acc_v[pl.ds(c * 16, 16)] = accs[c]
    pltpu.sync_copy(acc_v, out_hbm.at[wid])


@functools.lru_cache(maxsize=1)
def _build_embbag():
    return functools.partial(
        pl.kernel,
        out_type=jax.ShapeDtypeStruct((_NW, _OUTW), jnp.float32),
        mesh=plsc.VectorSubcoreMesh(core_axis_name="c", subcore_axis_name="s"),
        scratch_types=[
            pltpu.VMEM((_BPW,), jnp.int32),
            pltpu.VMEM((_BPW // 8, 128), jnp.float32),
            pltpu.VMEM((_BPW, 128), jnp.float32),
            pltpu.VMEM((_BPW, 128), jnp.float32),
            pltpu.VMEM((_BPW, 48), jnp.float32),
            pltpu.VMEM((_OUTW,), jnp.float32),
            pltpu.SemaphoreType.DMA,
            pltpu.SemaphoreType.DMA,
        ],
    )(_embbag_body)


def kernel(theta, bow, word_inputs, W1, b1, gamma, beta, W2, b2, embedding):
    inv = 1.0 / math.sqrt(1.0 + 1e-5)
    p = jnp.concatenate([theta, bow]).reshape(1, _K)
    scale = (gamma * inv).reshape(1, _HID)
    shift = (b1 * gamma * inv + beta).reshape(1, _HID)
    score = _mlp_call(p, W1.T, scale, shift, W2.reshape(1, _HID), b2.reshape(1, 1))

    pad = _PAD_N - _V_DIM
    idx_p = jnp.concatenate([word_inputs, jnp.zeros((pad,), jnp.int32)])
    w_p = jnp.concatenate([bow, jnp.zeros((pad,), jnp.float32)])
    w_p = jnp.broadcast_to(
        w_p.reshape(_PAD_N // 8, 8, 1), (_PAD_N // 8, 8, 16)
    ).reshape(_PAD_N // 8, 128)
    parts = _build_embbag()(idx_p, w_p, embedding)
    s = jnp.sum(parts, axis=0)
    return score.reshape(1), s[:_EMB]


# speedup vs baseline: 6.1709x; 2.1782x over previous
import functools
import math

import jax
import jax.numpy as jnp
from jax import lax
from jax.experimental import pallas as pl
from jax.experimental.pallas import tpu as pltpu
from jax.experimental.pallas import tpu_sc as plsc

_N_TOPIC = 256
_V_DIM = 10000
_HID = 2048
_EMB = 300
_VOCAB = 100000
_K = _N_TOPIC + _V_DIM

_BH = 256
_GRID = _HID // _BH


def _mlp_body(p_ref, w1t_ref, scale_ref, shift_ref, w2_ref, b2_ref, out_ref):
    i = pl.program_id(0)
    h = lax.dot_general(
        p_ref[...], w1t_ref[...], (((1,), (0,)), ((), ())),
        preferred_element_type=jnp.float32,
    )
    h = h * scale_ref[...] + shift_ref[...]
    h = jnp.where(h > 0.0, h, 0.01 * h)
    part = jnp.sum(h * w2_ref[...], axis=1, keepdims=True)

    @pl.when(i == 0)
    def _init():
        out_ref[...] = b2_ref[...] + part

    @pl.when(i > 0)
    def _acc():
        out_ref[...] += part


_mlp_call = pl.pallas_call(
    _mlp_body,
    grid=(_GRID,),
    in_specs=[
        pl.BlockSpec((1, _K), lambda i: (0, 0)),
        pl.BlockSpec((_K, _BH), lambda i: (0, i)),
        pl.BlockSpec((1, _BH), lambda i: (0, i)),
        pl.BlockSpec((1, _BH), lambda i: (0, i)),
        pl.BlockSpec((1, _BH), lambda i: (0, i)),
        pl.BlockSpec((1, 1), lambda i: (0, 0)),
    ],
    out_specs=pl.BlockSpec((1, 1), lambda i: (0, 0)),
    out_shape=jax.ShapeDtypeStruct((1, 1), jnp.float32),
)

_NW = 32
_BPW = 320
_PAD_N = _NW * _BPW
_GB = 80
_GCH = _BPW // _GB
_VPAD = 102400
_STRIPE = _VPAD // 16


def _scatter_body(idx_hbm, w_hbm, out_hbm, idx_v, w_v, zero_v, s_sh, sem):
    cid = lax.axis_index("c")
    sid = lax.axis_index("s")
    wid = sid * 2 + cid
    base = wid * _BPW
    for g in range(_GCH):
        pltpu.sync_copy(idx_hbm.at[pl.ds(base + g * _GB, _GB)], idx_v.at[g])
        pltpu.sync_copy(w_hbm.at[pl.ds(base + g * _GB, _GB)], w_v.at[g])

    def zloop(j, carry):
        zero_v[pl.ds(j * 16, 16)] = jnp.zeros((16,), jnp.float32)
        return carry

    lax.fori_loop(0, _STRIPE // 16, zloop, 0)
    pltpu.sync_copy(zero_v, s_sh.at[pl.ds(sid * _STRIPE, _STRIPE)])

    plsc.subcore_barrier()

    for g in range(_GCH):
        pltpu.sync_copy(w_v.at[g], s_sh.at[idx_v.at[g]], add=True)

    plsc.subcore_barrier()

    pltpu.sync_copy(s_sh.at[pl.ds(sid * _STRIPE, _STRIPE)],
                    out_hbm.at[cid, pl.ds(sid * _STRIPE, _STRIPE)])


@functools.lru_cache(maxsize=1)
def _build_scatter():
    return functools.partial(
        pl.kernel,
        out_type=jax.ShapeDtypeStruct((2, _VPAD), jnp.float32),
        mesh=plsc.VectorSubcoreMesh(core_axis_name="c", subcore_axis_name="s"),
        scratch_types=[
            pltpu.VMEM((_GCH, _GB), jnp.int32),
            pltpu.VMEM((_GCH, _GB), jnp.float32),
            pltpu.VMEM((_STRIPE,), jnp.float32),
            pltpu.VMEM_SHARED((_VPAD,), jnp.float32),
            pltpu.SemaphoreType.DMA,
        ],
    )(_scatter_body)


_BV = 6400
_VGRID = _VPAD // _BV


def _bag_body(s_ref, et_ref, out_ref):
    k = pl.program_id(0)
    sv = s_ref[0:1, :] + s_ref[1:2, :]
    last_valid = _VOCAB - (_VGRID - 1) * _BV

    @pl.when(k == _VGRID - 1)
    def _mask():
        col = lax.broadcasted_iota(jnp.int32, (_EMB, _BV), 1)
        et_ref[...] = jnp.where(col < last_valid, et_ref[...], 0.0)

    et = et_ref[...]
    part = lax.dot_general(
        sv, et, (((1,), (1,)), ((), ())),
        preferred_element_type=jnp.float32,
    )

    @pl.when(k == 0)
    def _init():
        out_ref[...] = part

    @pl.when(k > 0)
    def _acc():
        out_ref[...] += part


_bag_call = pl.pallas_call(
    _bag_body,
    grid=(_VGRID,),
    in_specs=[
        pl.BlockSpec((2, _BV), lambda k: (0, k)),
        pl.BlockSpec((_EMB, _BV), lambda k: (0, k)),
    ],
    out_specs=pl.BlockSpec((1, _EMB), lambda k: (0, 0)),
    out_shape=jax.ShapeDtypeStruct((1, _EMB), jnp.float32),
)


def kernel(theta, bow, word_inputs, W1, b1, gamma, beta, W2, b2, embedding):
    inv = 1.0 / math.sqrt(1.0 + 1e-5)
    p = jnp.concatenate([theta, bow]).reshape(1, _K)
    scale = (gamma * inv).reshape(1, _HID)
    shift = (b1 * gamma * inv + beta).reshape(1, _HID)
    score = _mlp_call(p, W1.T, scale, shift, W2.reshape(1, _HID), b2.reshape(1, 1))

    pad = _PAD_N - _V_DIM
    idx_p = jnp.concatenate([word_inputs, jnp.zeros((pad,), jnp.int32)])
    w_p = jnp.concatenate([bow, jnp.zeros((pad,), jnp.float32)])
    s2 = _build_scatter()(idx_p, w_p)
    word_embedding = _bag_call(s2, embedding.T).reshape(_EMB)
    return score.reshape(1), word_embedding
